# baseline (speedup 1.0000x reference)
import functools

import jax
import jax.numpy as jnp
from jax import lax
from jax.experimental import pallas as pl
from jax.experimental.pallas import tpu as pltpu
from jax.experimental.pallas import tpu_sc as plsc

V = 100000
D = 128
H = 256
NP = 1000
NPP = 1024
B = 4096
GROUP = 200
HALF = GROUP // 2
NSLOT = 4
RUNROLL = 10
OSTAGE = 16


def _make_pool_kernel():
    info = plsc.get_sparse_core_info()
    nc, ns = info.num_cores, info.num_subcores
    nw = nc * ns
    assert B % nw == 0
    bpw = B // nw

    mesh = plsc.VectorSubcoreMesh(core_axis_name="c", subcore_axis_name="s")

    @functools.partial(
        pl.kernel,
        mesh=mesh,
        out_type=jax.ShapeDtypeStruct((B, D), jnp.float32),
        scratch_types=[
            pltpu.VMEM((B // nw // 2, 2, HALF), jnp.int32),
            pltpu.VMEM((NSLOT, 2, HALF), jnp.int32),
            pltpu.VMEM((NSLOT, GROUP, D), jnp.float32),
            pltpu.VMEM((OSTAGE, D), jnp.float32),
            pltpu.SemaphoreType.DMA,
            pltpu.SemaphoreType.DMA,
            pltpu.SemaphoreType.DMA,
            pltpu.SemaphoreType.DMA,
        ],
    )
    def pool(idx_hbm, table_hbm, out_hbm, idx_v, pre_v, rows_v, out_v, *sems):
        wid = lax.axis_index("s") * nc + lax.axis_index("c")
        base = wid * bpw
        hb = bpw // 2

        def fire(iref, row, slot):
            pltpu.async_copy(table_hbm.at[iref.at[row, 0]],
                             rows_v.at[slot, pl.ds(0, HALF)], sems[slot])
            pltpu.async_copy(table_hbm.at[iref.at[row, 1]],
                             rows_v.at[slot, pl.ds(HALF, HALF)], sems[slot])

        def drain(slot):
            pltpu.make_async_copy(table_hbm.at[pl.ds(0, GROUP)],
                                  rows_v.at[slot], sems[slot]).wait()

        pltpu.sync_copy(idx_hbm.at[pl.ds(base, hb)], idx_v)
        for s in range(NSLOT):
            fire(idx_v, s, s)

        assert hb % OSTAGE == 0 and OSTAGE % NSLOT == 0

        for h in range(2):
            if h == 0:
                pltpu.sync_copy(idx_hbm.at[pl.ds(base + hb, NSLOT)], pre_v)

            def body(i, _):
                row = i * NSLOT
                for slot in range(NSLOT):
                    drain(slot)

                    def rbody(r8, accs):
                        r = r8 * RUNROLL
                        new = list(accs)
                        for dr in range(RUNROLL):
                            for k in range(8):
                                new[k] = new[k] + rows_v[slot, r + dr,
                                                         pl.ds(k * 16, 16)]
                        return tuple(new)

                    accs = lax.fori_loop(
                        0, GROUP // RUNROLL, rbody,
                        tuple(jnp.zeros((16,), jnp.float32)
                              for _ in range(8)),
                        unroll=1)
                    orow = (row + slot) % OSTAGE
                    for k in range(8):
                        out_v[orow, pl.ds(k * 16, 16)] = accs[k]

                    nxt = row + slot + NSLOT

                    @pl.when(nxt < hb)
                    def _():
                        fire(idx_v, nxt, slot)

                    if h == 0:
                        @pl.when(nxt >= hb)
                        def _():
                            fire(pre_v, nxt - hb, slot)

                @pl.when((row + NSLOT) % OSTAGE == 0)
                def _():
                    off = pl.multiple_of(
                        base + h * hb + row + NSLOT - OSTAGE, OSTAGE)
                    pltpu.sync_copy(out_v, out_hbm.at[pl.ds(off, OSTAGE)])
                return 0

            lax.fori_loop(0, hb // NSLOT, body, 0)
            if h == 0:
                pltpu.sync_copy(idx_hbm.at[pl.ds(base + hb, hb)], idx_v)

    return pool


def _mlp_body(s_ref, w1_ref, b1_ref, w2_ref, b2_ref, out_ref):
    s = s_ref[...]
    h = jax.nn.sigmoid(
        jnp.dot(s, w1_ref[...], preferred_element_type=jnp.float32)
        + b1_ref[...])
    logits = (jnp.dot(h, w2_ref[...], preferred_element_type=jnp.float32)
              + b2_ref[...])
    m = jnp.max(logits, axis=-1, keepdims=True)
    lse = jnp.log(jnp.sum(jnp.exp(logits - m), axis=-1, keepdims=True)) + m
    out_ref[...] = logits - lse


def _mlp(pooled, w1, b1, w2, b2):
    bm = 512
    grid = (B // bm,)
    return pl.pallas_call(
        _mlp_body,
        grid=grid,
        in_specs=[
            pl.BlockSpec((bm, D), lambda i: (i, 0)),
            pl.BlockSpec((D, H), lambda i: (0, 0)),
            pl.BlockSpec((1, H), lambda i: (0, 0)),
            pl.BlockSpec((H, NPP), lambda i: (0, 0)),
            pl.BlockSpec((1, NPP), lambda i: (0, 0)),
        ],
        out_specs=pl.BlockSpec((bm, NPP), lambda i: (i, 0)),
        out_shape=jax.ShapeDtypeStruct((B, NPP), jnp.float32),
    )(pooled, w1, b1, w2, b2)


def kernel(x, table, W1, b1, W2, b2):
    idx = x.reshape(B, 2, HALF)
    pooled = _make_pool_kernel()(idx, table)
    w2p = jnp.pad(W2, ((0, 0), (0, NPP - NP)))
    b2p = jnp.pad(b2, (0, NPP - NP), constant_values=-1e30)
    out = _mlp(pooled, W1, b1.reshape(1, H), w2p, b2p.reshape(1, NPP))
    return out[:, :NP]


# speedup vs baseline: 1.0155x; 1.0155x over previous
import functools

import jax
import jax.numpy as jnp
from jax import lax
from jax.experimental import pallas as pl
from jax.experimental.pallas import tpu as pltpu
from jax.experimental.pallas import tpu_sc as plsc

V = 100000
D = 128
H = 256
NP = 1000
NPP = 1024
B = 4096
GROUP = 200
HALF = GROUP // 2
NSLOT = 3
RUNROLL = 8


def _make_pool_kernel():
    info = plsc.get_sparse_core_info()
    nc, ns = info.num_cores, info.num_subcores
    nw = nc * ns
    assert B % nw == 0
    bpw = B // nw

    mesh = plsc.VectorSubcoreMesh(core_axis_name="c", subcore_axis_name="s")

    @functools.partial(
        pl.kernel,
        mesh=mesh,
        out_type=jax.ShapeDtypeStruct((B, D), jnp.float32),
        scratch_types=[
            pltpu.VMEM((bpw, 2, HALF), jnp.int32),
            pltpu.VMEM((NSLOT, GROUP, D), jnp.float32),
            pltpu.VMEM((bpw, D), jnp.float32),
            pltpu.SemaphoreType.DMA,
            pltpu.SemaphoreType.DMA,
            pltpu.SemaphoreType.DMA,
        ],
    )
    def pool(idx_hbm, table_hbm, out_hbm, idx_v, rows_v, out_v, *sems):
        wid = lax.axis_index("s") * nc + lax.axis_index("c")
        base = wid * bpw

        pltpu.sync_copy(idx_hbm.at[pl.ds(base, bpw)], idx_v)

        def fire(row, slot):
            pltpu.async_copy(table_hbm.at[idx_v.at[row, 0]],
                             rows_v.at[slot, pl.ds(0, HALF)], sems[slot])
            pltpu.async_copy(table_hbm.at[idx_v.at[row, 1]],
                             rows_v.at[slot, pl.ds(HALF, HALF)], sems[slot])

        def drain(row, slot):
            pltpu.make_async_copy(table_hbm.at[idx_v.at[row, 0]],
                                  rows_v.at[slot, pl.ds(0, HALF)],
                                  sems[slot]).wait()
            pltpu.make_async_copy(table_hbm.at[idx_v.at[row, 1]],
                                  rows_v.at[slot, pl.ds(HALF, HALF)],
                                  sems[slot]).wait()

        for s in range(NSLOT):
            fire(s, s)

        def body(i, _):
            row = i * NSLOT
            for slot in range(NSLOT):
                drain(row + slot, slot)

                def rbody(r8, accs):
                    r = r8 * RUNROLL
                    new = list(accs)
                    for dr in range(RUNROLL):
                        for k in range(8):
                            new[k] = new[k] + rows_v[slot, r + dr,
                                                     pl.ds(k * 16, 16)]
                    return tuple(new)

                accs = lax.fori_loop(
                    0, GROUP // RUNROLL, rbody,
                    tuple(jnp.zeros((16,), jnp.float32) for _ in range(8)),
                    unroll=1)
                for k in range(8):
                    out_v[row + slot, pl.ds(k * 16, 16)] = accs[k]

                @pl.when(row + slot + NSLOT < bpw)
                def _():
                    fire(row + slot + NSLOT, slot)
            return 0

        nfull = bpw // NSLOT
        lax.fori_loop(0, nfull, body, 0)
        for t in range(nfull * NSLOT, bpw):
            slot = t % NSLOT
            drain(t, slot)

            def rtail(r8, accs):
                r = r8 * RUNROLL
                new = list(accs)
                for dr in range(RUNROLL):
                    for k in range(8):
                        new[k] = new[k] + rows_v[slot, r + dr,
                                                 pl.ds(k * 16, 16)]
                return tuple(new)

            accs = lax.fori_loop(
                0, GROUP // RUNROLL, rtail,
                tuple(jnp.zeros((16,), jnp.float32) for _ in range(8)),
                unroll=1)
            for k in range(8):
                out_v[t, pl.ds(k * 16, 16)] = accs[k]

        pltpu.sync_copy(out_v, out_hbm.at[pl.ds(base, bpw)])

    return pool


def _mlp_body(s_ref, w1_ref, b1_ref, w2_ref, b2_ref, out_ref):
    s = s_ref[...]
    h = jax.nn.sigmoid(
        jnp.dot(s, w1_ref[...], preferred_element_type=jnp.float32)
        + b1_ref[...])
    logits = (jnp.dot(h, w2_ref[...], preferred_element_type=jnp.float32)
              + b2_ref[...])
    m = jnp.max(logits, axis=-1, keepdims=True)
    lse = jnp.log(jnp.sum(jnp.exp(logits - m), axis=-1, keepdims=True)) + m
    out_ref[...] = logits - lse


def _mlp(pooled, w1, b1, w2, b2):
    bm = 512
    grid = (B // bm,)
    return pl.pallas_call(
        _mlp_body,
        grid=grid,
        in_specs=[
            pl.BlockSpec((bm, D), lambda i: (i, 0)),
            pl.BlockSpec((D, H), lambda i: (0, 0)),
            pl.BlockSpec((1, H), lambda i: (0, 0)),
            pl.BlockSpec((H, NPP), lambda i: (0, 0)),
            pl.BlockSpec((1, NPP), lambda i: (0, 0)),
        ],
        out_specs=pl.BlockSpec((bm, NPP), lambda i: (i, 0)),
        out_shape=jax.ShapeDtypeStruct((B, NPP), jnp.float32),
    )(pooled, w1, b1, w2, b2)


def kernel(x, table, W1, b1, W2, b2):
    idx = x.reshape(B, 2, HALF)
    pooled = _make_pool_kernel()(idx, table)
    w2p = jnp.pad(W2, ((0, 0), (0, NPP - NP)))
    b2p = jnp.pad(b2, (0, NPP - NP), constant_values=-1e30)
    out = _mlp(pooled, W1, b1.reshape(1, H), w2p, b2p.reshape(1, NPP))
    return out[:, :NP]


# speedup vs baseline: 1.0245x; 1.0089x over previous
import functools

import jax
import jax.numpy as jnp
from jax import lax
from jax.experimental import pallas as pl
from jax.experimental.pallas import tpu as pltpu
from jax.experimental.pallas import tpu_sc as plsc

V = 100000
D = 128
H = 256
NP = 1000
NPP = 1024
B = 4096
GROUP = 200
HALF = GROUP // 2
NSLOT = 3
RUNROLL = 8


def _make_pool_kernel():
    info = plsc.get_sparse_core_info()
    nc, ns = info.num_cores, info.num_subcores
    nw = nc * ns
    assert B % nw == 0
    bpw = B // nw

    mesh = plsc.VectorSubcoreMesh(core_axis_name="c", subcore_axis_name="s")

    @functools.partial(
        pl.kernel,
        mesh=mesh,
        out_type=jax.ShapeDtypeStruct((B, D), jnp.float32),
        scratch_types=[
            pltpu.VMEM((bpw, 2, HALF), jnp.int32),
            pltpu.VMEM((NSLOT, GROUP, D), jnp.float32),
            pltpu.VMEM((bpw, D), jnp.float32),
            pltpu.SemaphoreType.DMA,
            pltpu.SemaphoreType.DMA,
            pltpu.SemaphoreType.DMA,
        ],
    )
    def pool(idx_hbm, table_hbm, out_hbm, idx_v, rows_v, out_v, *sems):
        wid = lax.axis_index("s") * nc + lax.axis_index("c")
        base = wid * bpw

        pltpu.sync_copy(idx_hbm.at[pl.ds(base, bpw)], idx_v)

        def fire(row, slot):
            pltpu.async_copy(table_hbm.at[idx_v.at[row, 0]],
                             rows_v.at[slot, pl.ds(0, HALF)], sems[slot])
            pltpu.async_copy(table_hbm.at[idx_v.at[row, 1]],
                             rows_v.at[slot, pl.ds(HALF, HALF)], sems[slot])

        def drain(row, slot):
            pltpu.make_async_copy(table_hbm.at[idx_v.at[row, 0]],
                                  rows_v.at[slot, pl.ds(0, HALF)],
                                  sems[slot]).wait()
            pltpu.make_async_copy(table_hbm.at[idx_v.at[row, 1]],
                                  rows_v.at[slot, pl.ds(HALF, HALF)],
                                  sems[slot]).wait()

        for s in range(NSLOT):
            fire(s, s)

        def body(i, _):
            row = i * NSLOT
            for slot in range(NSLOT):
                drain(row + slot, slot)

                def rbody(r8, accs):
                    r = r8 * RUNROLL
                    new = list(accs)
                    for dr in range(RUNROLL):
                        for k in range(8):
                            new[k] = new[k] + rows_v[slot, r + dr,
                                                     pl.ds(k * 16, 16)]
                    return tuple(new)

                accs = lax.fori_loop(
                    0, GROUP // RUNROLL, rbody,
                    tuple(jnp.zeros((16,), jnp.float32) for _ in range(8)),
                    unroll=1)
                for k in range(8):
                    out_v[row + slot, pl.ds(k * 16, 16)] = accs[k]

                @pl.when(row + slot + NSLOT < bpw)
                def _():
                    fire(row + slot + NSLOT, slot)
            return 0

        nfull = bpw // NSLOT
        lax.fori_loop(0, nfull, body, 0)
        for t in range(nfull * NSLOT, bpw):
            slot = t % NSLOT
            drain(t, slot)

            def rtail(r8, accs):
                r = r8 * RUNROLL
                new = list(accs)
                for dr in range(RUNROLL):
                    for k in range(8):
                        new[k] = new[k] + rows_v[slot, r + dr,
                                                 pl.ds(k * 16, 16)]
                return tuple(new)

            accs = lax.fori_loop(
                0, GROUP // RUNROLL, rtail,
                tuple(jnp.zeros((16,), jnp.float32) for _ in range(8)),
                unroll=1)
            for k in range(8):
                out_v[t, pl.ds(k * 16, 16)] = accs[k]

        pltpu.sync_copy(out_v, out_hbm.at[pl.ds(base, bpw)])

    return pool


def _mlp_body(s_ref, w1_ref, b1_ref, w2_ref, b2_ref, out_ref):
    s = s_ref[...]
    h = jax.nn.sigmoid(
        jnp.dot(s, w1_ref[...], preferred_element_type=jnp.float32)
        + b1_ref[...])
    logits = (jnp.dot(h, w2_ref[...], preferred_element_type=jnp.float32)
              + b2_ref[...])
    m = jnp.max(logits, axis=-1, keepdims=True)
    lse = jnp.log(jnp.sum(jnp.exp(logits - m), axis=-1, keepdims=True)) + m
    out_ref[...] = logits - lse


def _mlp(pooled, w1, b1, w2, b2):
    bm = 1024
    grid = (B // bm,)
    return pl.pallas_call(
        _mlp_body,
        grid=grid,
        in_specs=[
            pl.BlockSpec((bm, D), lambda i: (i, 0)),
            pl.BlockSpec((D, H), lambda i: (0, 0)),
            pl.BlockSpec((1, H), lambda i: (0, 0)),
            pl.BlockSpec((H, NPP), lambda i: (0, 0)),
            pl.BlockSpec((1, NPP), lambda i: (0, 0)),
        ],
        out_specs=pl.BlockSpec((bm, NPP), lambda i: (i, 0)),
        out_shape=jax.ShapeDtypeStruct((B, NPP), jnp.float32),
    )(pooled, w1, b1, w2, b2)


def kernel(x, table, W1, b1, W2, b2):
    idx = x.reshape(B, 2, HALF)
    pooled = _make_pool_kernel()(idx, table)
    w2p = jnp.pad(W2, ((0, 0), (0, NPP - NP)))
    b2p = jnp.pad(b2, (0, NPP - NP), constant_values=-1e30)
    out = _mlp(pooled, W1, b1.reshape(1, H), w2p, b2p.reshape(1, NPP))
    return out[:, :NP]


# speedup vs baseline: 1.0247x; 1.0002x over previous
import functools

import jax
import jax.numpy as jnp
from jax import lax
from jax.experimental import pallas as pl
from jax.experimental.pallas import tpu as pltpu
from jax.experimental.pallas import tpu_sc as plsc

V = 100000
D = 128
H = 256
NP = 1000
NPP = 1024
B = 4096
GROUP = 200
HALF = GROUP // 2
NSLOT = 3
RUNROLL = 8


def _make_pool_kernel():
    info = plsc.get_sparse_core_info()
    nc, ns = info.num_cores, info.num_subcores
    nw = nc * ns
    assert B % nw == 0
    bpw = B // nw

    mesh = plsc.VectorSubcoreMesh(core_axis_name="c", subcore_axis_name="s")

    @functools.partial(
        pl.kernel,
        mesh=mesh,
        out_type=jax.ShapeDtypeStruct((B, D), jnp.float32),
        scratch_types=[
            pltpu.VMEM((bpw, 2, HALF), jnp.int32),
            pltpu.VMEM((NSLOT, GROUP, D), jnp.float32),
            pltpu.VMEM((bpw, D), jnp.float32),
            pltpu.SemaphoreType.DMA,
            pltpu.SemaphoreType.DMA,
            pltpu.SemaphoreType.DMA,
        ],
    )
    def pool(idx_hbm, table_hbm, out_hbm, idx_v, rows_v, out_v, *sems):
        wid = lax.axis_index("s") * nc + lax.axis_index("c")
        base = wid * bpw

        pltpu.sync_copy(idx_hbm.at[pl.ds(base, bpw)], idx_v)

        def fire(row, slot):
            pltpu.async_copy(table_hbm.at[idx_v.at[row, 0]],
                             rows_v.at[slot, pl.ds(0, HALF)], sems[slot])
            pltpu.async_copy(table_hbm.at[idx_v.at[row, 1]],
                             rows_v.at[slot, pl.ds(HALF, HALF)], sems[slot])

        def drain(row, slot):
            pltpu.make_async_copy(table_hbm.at[idx_v.at[row, 0]],
                                  rows_v.at[slot, pl.ds(0, HALF)],
                                  sems[slot]).wait()
            pltpu.make_async_copy(table_hbm.at[idx_v.at[row, 1]],
                                  rows_v.at[slot, pl.ds(HALF, HALF)],
                                  sems[slot]).wait()

        for s in range(NSLOT):
            fire(s, s)

        def body(i, _):
            row = i * NSLOT
            for slot in range(NSLOT):
                drain(row + slot, slot)

                def rbody(r8, accs):
                    r = r8 * RUNROLL
                    new = list(accs)
                    for dr in range(RUNROLL):
                        for k in range(8):
                            new[k] = new[k] + rows_v[slot, r + dr,
                                                     pl.ds(k * 16, 16)]
                    return tuple(new)

                accs = lax.fori_loop(
                    0, GROUP // RUNROLL, rbody,
                    tuple(jnp.zeros((16,), jnp.float32) for _ in range(8)),
                    unroll=1)
                for k in range(8):
                    out_v[row + slot, pl.ds(k * 16, 16)] = accs[k]

                @pl.when(row + slot + NSLOT < bpw)
                def _():
                    fire(row + slot + NSLOT, slot)
            return 0

        nfull = bpw // NSLOT
        lax.fori_loop(0, nfull, body, 0)
        for t in range(nfull * NSLOT, bpw):
            slot = t % NSLOT
            drain(t, slot)

            def rtail(r8, accs):
                r = r8 * RUNROLL
                new = list(accs)
                for dr in range(RUNROLL):
                    for k in range(8):
                        new[k] = new[k] + rows_v[slot, r + dr,
                                                 pl.ds(k * 16, 16)]
                return tuple(new)

            accs = lax.fori_loop(
                0, GROUP // RUNROLL, rtail,
                tuple(jnp.zeros((16,), jnp.float32) for _ in range(8)),
                unroll=1)
            for k in range(8):
                out_v[t, pl.ds(k * 16, 16)] = accs[k]

        pltpu.sync_copy(out_v, out_hbm.at[pl.ds(base, bpw)])

    return pool


def _mlp_body(s_ref, w1_ref, b1_ref, w2_ref, b2_ref, out_ref):
    s = s_ref[...]
    h = jax.nn.sigmoid(
        jnp.dot(s, w1_ref[...], preferred_element_type=jnp.float32)
        + b1_ref[...])
    logits = (jnp.dot(h, w2_ref[...], preferred_element_type=jnp.float32)
              + b2_ref[...])
    m = jnp.max(logits, axis=-1, keepdims=True)
    lse = jnp.log(jnp.sum(jnp.exp(logits - m), axis=-1, keepdims=True)) + m
    out_ref[...] = logits - lse


def _mlp(pooled, w1, b1, w2, b2):
    bm = 2048
    grid = (B // bm,)
    return pl.pallas_call(
        _mlp_body,
        grid=grid,
        in_specs=[
            pl.BlockSpec((bm, D), lambda i: (i, 0)),
            pl.BlockSpec((D, H), lambda i: (0, 0)),
            pl.BlockSpec((1, H), lambda i: (0, 0)),
            pl.BlockSpec((H, NPP), lambda i: (0, 0)),
            pl.BlockSpec((1, NPP), lambda i: (0, 0)),
        ],
        out_specs=pl.BlockSpec((bm, NPP), lambda i: (i, 0)),
        out_shape=jax.ShapeDtypeStruct((B, NPP), jnp.float32),
    )(pooled, w1, b1, w2, b2)


def kernel(x, table, W1, b1, W2, b2):
    idx = x.reshape(B, 2, HALF)
    pooled = _make_pool_kernel()(idx, table)
    w2p = jnp.pad(W2, ((0, 0), (0, NPP - NP)))
    b2p = jnp.pad(b2, (0, NPP - NP), constant_values=-1e30)
    out = _mlp(pooled, W1, b1.reshape(1, H), w2p, b2p.reshape(1, NPP))
    return out[:, :NP]
